# baseline (speedup 1.0000x reference)
import functools

import jax
import jax.numpy as jnp
from jax import lax
from jax.experimental import pallas as pl
from jax.experimental.pallas import tpu as pltpu
from jax.experimental.pallas import tpu_sc as plsc

B = 16384
NF = 64
H1 = 256

NC = 2
NS = 16
NW = NC * NS
BPW = B // NW


def _sc_gather(users2, items2, user_table, item_table):
    mesh = plsc.VectorSubcoreMesh(core_axis_name="c", subcore_axis_name="s")

    @functools.partial(
        pl.kernel,
        out_type=(
            jax.ShapeDtypeStruct((B, NF), jnp.float32),
            jax.ShapeDtypeStruct((B, NF), jnp.float32),
        ),
        mesh=mesh,
        scratch_types=[
            pltpu.VMEM((BPW,), jnp.int32),
            pltpu.VMEM((BPW,), jnp.int32),
            pltpu.SemaphoreType.DMA,
        ],
        compiler_params=pltpu.CompilerParams(needs_layout_passes=False),
    )
    def k(users_hbm, items_hbm, ut_hbm, it_hbm, u_out, i_out,
          idx_u, idx_i, sem):
        wid = lax.axis_index("s") * NC + lax.axis_index("c")
        base = wid * BPW
        pltpu.sync_copy(users_hbm.at[wid], idx_u)
        pltpu.sync_copy(items_hbm.at[wid], idx_i)
        lanes = lax.iota(jnp.int32, 16)

        def fire(idx_ref, table_hbm, out_hbm):
            def group(t, _):
                v16 = idx_ref[pl.ds(t * 16, 16)]
                for l in range(16):
                    s = jnp.max(jnp.where(lanes == l, v16, 0))
                    pltpu.async_copy(
                        table_hbm.at[s], out_hbm.at[base + t * 16 + l], sem)
                return 0

            lax.fori_loop(0, BPW // 16, group, 0)

        fire(idx_u, ut_hbm, u_out)
        fire(idx_i, it_hbm, i_out)

        def drain(j, _):
            pltpu.make_async_copy(ut_hbm.at[0], u_out.at[base], sem).wait()
            return 0

        lax.fori_loop(0, 2 * BPW, drain, 0)

    return k(users2, items2, user_table, item_table)


BS = 2048


def _mlp_body(u_ref, i_ref, w1u_ref, w1i_ref, b1_ref, w2t_ref, b2_ref, o_ref):
    h = (
        jnp.dot(u_ref[...], w1u_ref[...], preferred_element_type=jnp.float32)
        + jnp.dot(i_ref[...], w1i_ref[...], preferred_element_type=jnp.float32)
        + b1_ref[...]
    )
    h = jnp.maximum(h, 0.0)
    o_ref[...] = jnp.sum(h * w2t_ref[...], axis=1, keepdims=True) + b2_ref[...]


def _mlp(u_emb, i_emb, W1u, W1i, b1, W2t, b2):
    return pl.pallas_call(
        _mlp_body,
        grid=(B // BS,),
        in_specs=[
            pl.BlockSpec((BS, NF), lambda i: (i, 0)),
            pl.BlockSpec((BS, NF), lambda i: (i, 0)),
            pl.BlockSpec((NF, H1), lambda i: (0, 0)),
            pl.BlockSpec((NF, H1), lambda i: (0, 0)),
            pl.BlockSpec((1, H1), lambda i: (0, 0)),
            pl.BlockSpec((1, H1), lambda i: (0, 0)),
            pl.BlockSpec((1, 1), lambda i: (0, 0)),
        ],
        out_specs=pl.BlockSpec((BS, 1), lambda i: (i, 0)),
        out_shape=jax.ShapeDtypeStruct((B, 1), jnp.float32),
    )(u_emb, i_emb, W1u, W1i, b1, W2t, b2)


@jax.jit
def kernel(users, items, user_table, item_table, W1, b1, W2, b2):
    users2 = users.reshape(NW, BPW)
    items2 = items.reshape(NW, BPW)
    u_emb, i_emb = _sc_gather(users2, items2, user_table, item_table)
    W1u = W1[:NF]
    W1i = W1[NF:]
    return _mlp(u_emb, i_emb, W1u, W1i,
                b1.reshape(1, H1), W2.reshape(1, H1), b2.reshape(1, 1))


# speedup vs baseline: 1.6771x; 1.6771x over previous
import functools

import jax
import jax.numpy as jnp
from jax import lax
from jax.experimental import pallas as pl
from jax.experimental.pallas import tpu as pltpu
from jax.experimental.pallas import tpu_sc as plsc

B = 16384
NF = 64
H1 = 256

NC = 2
NS = 16
NW = NC * NS
BPW = B // NW


def _sc_gather(users2, items2, user_table, item_table):
    mesh = plsc.VectorSubcoreMesh(core_axis_name="c", subcore_axis_name="s")

    @functools.partial(
        pl.kernel,
        out_type=(
            jax.ShapeDtypeStruct((B, NF), jnp.float32),
            jax.ShapeDtypeStruct((B, NF), jnp.float32),
        ),
        mesh=mesh,
        scratch_types=[
            pltpu.VMEM((BPW,), jnp.int32),
            pltpu.VMEM((BPW,), jnp.int32),
            pltpu.VMEM((BPW, NF), jnp.float32),
            pltpu.SemaphoreType.DMA,
        ],
        compiler_params=pltpu.CompilerParams(needs_layout_passes=False),
    )
    def k(users_hbm, items_hbm, ut_hbm, it_hbm, u_out, i_out,
          idx_u, idx_i, rows, sem):
        wid = lax.axis_index("s") * NC + lax.axis_index("c")
        base = wid * BPW
        pltpu.sync_copy(users_hbm.at[wid], idx_u)
        pltpu.sync_copy(items_hbm.at[wid], idx_i)
        lanes = lax.iota(jnp.int32, 16)

        def one_table(idx_ref, table_hbm, out_hbm):
            def group(t, _):
                v16 = idx_ref[pl.ds(t * 16, 16)]
                for l in range(16):
                    s = jnp.max(jnp.where(lanes == l, v16, 0))
                    pltpu.async_copy(
                        table_hbm.at[s], rows.at[t * 16 + l], sem)
                return 0

            lax.fori_loop(0, BPW // 16, group, 0)

            def drain(j, _):
                pltpu.make_async_copy(table_hbm.at[0], rows.at[0], sem).wait()
                return 0

            lax.fori_loop(0, BPW, drain, 0)
            pltpu.sync_copy(rows, out_hbm.at[pl.ds(base, BPW)])

        one_table(idx_u, ut_hbm, u_out)
        one_table(idx_i, it_hbm, i_out)

    return k(users2, items2, user_table, item_table)


BS = 2048


def _mlp_body(u_ref, i_ref, w1u_ref, w1i_ref, b1_ref, w2t_ref, b2_ref, o_ref):
    h = (
        jnp.dot(u_ref[...], w1u_ref[...], preferred_element_type=jnp.float32)
        + jnp.dot(i_ref[...], w1i_ref[...], preferred_element_type=jnp.float32)
        + b1_ref[...]
    )
    h = jnp.maximum(h, 0.0)
    o_ref[...] = jnp.sum(h * w2t_ref[...], axis=1, keepdims=True) + b2_ref[...]


def _mlp(u_emb, i_emb, W1u, W1i, b1, W2t, b2):
    return pl.pallas_call(
        _mlp_body,
        grid=(B // BS,),
        in_specs=[
            pl.BlockSpec((BS, NF), lambda i: (i, 0)),
            pl.BlockSpec((BS, NF), lambda i: (i, 0)),
            pl.BlockSpec((NF, H1), lambda i: (0, 0)),
            pl.BlockSpec((NF, H1), lambda i: (0, 0)),
            pl.BlockSpec((1, H1), lambda i: (0, 0)),
            pl.BlockSpec((1, H1), lambda i: (0, 0)),
            pl.BlockSpec((1, 1), lambda i: (0, 0)),
        ],
        out_specs=pl.BlockSpec((BS, 1), lambda i: (i, 0)),
        out_shape=jax.ShapeDtypeStruct((B, 1), jnp.float32),
    )(u_emb, i_emb, W1u, W1i, b1, W2t, b2)


@jax.jit
def kernel(users, items, user_table, item_table, W1, b1, W2, b2):
    users2 = users.reshape(NW, BPW)
    items2 = items.reshape(NW, BPW)
    u_emb, i_emb = _sc_gather(users2, items2, user_table, item_table)
    W1u = W1[:NF]
    W1i = W1[NF:]
    return _mlp(u_emb, i_emb, W1u, W1i,
                b1.reshape(1, H1), W2.reshape(1, H1), b2.reshape(1, 1))


# speedup vs baseline: 1.6805x; 1.0021x over previous
import functools

import jax
import jax.numpy as jnp
from jax import lax
from jax.experimental import pallas as pl
from jax.experimental.pallas import tpu as pltpu
from jax.experimental.pallas import tpu_sc as plsc

B = 16384
NF = 64
H1 = 256

NC = 2
NS = 16
NW = NC * NS
BPW = B // NW


def _sc_gather(users2, items2, user_table, item_table):
    mesh = plsc.VectorSubcoreMesh(core_axis_name="c", subcore_axis_name="s")

    @functools.partial(
        pl.kernel,
        out_type=(
            jax.ShapeDtypeStruct((B, NF), jnp.float32),
            jax.ShapeDtypeStruct((B, NF), jnp.float32),
        ),
        mesh=mesh,
        scratch_types=[
            pltpu.VMEM((BPW,), jnp.int32),
            pltpu.VMEM((BPW,), jnp.int32),
            pltpu.VMEM((BPW, NF), jnp.float32),
            pltpu.SemaphoreType.DMA,
        ],
    )
    def k(users_hbm, items_hbm, ut_hbm, it_hbm, u_out, i_out,
          idx_u, idx_i, rows, sem):
        wid = lax.axis_index("s") * NC + lax.axis_index("c")
        base = wid * BPW
        pltpu.sync_copy(users_hbm.at[wid], idx_u)
        pltpu.sync_copy(items_hbm.at[wid], idx_i)
        lanes = lax.iota(jnp.int32, 16)

        def one_table(idx_ref, table_hbm, out_hbm):
            def group(t, _):
                v16 = idx_ref[pl.ds(t * 16, 16)]
                for l in range(16):
                    s = v16[l]
                    pltpu.async_copy(
                        table_hbm.at[s], rows.at[t * 16 + l], sem)
                return 0

            lax.fori_loop(0, BPW // 16, group, 0)

            def drain(j, _):
                pltpu.make_async_copy(table_hbm.at[0], rows.at[0], sem).wait()
                return 0

            lax.fori_loop(0, BPW, drain, 0)
            pltpu.sync_copy(rows, out_hbm.at[pl.ds(base, BPW)])

        one_table(idx_u, ut_hbm, u_out)
        one_table(idx_i, it_hbm, i_out)

    return k(users2, items2, user_table, item_table)


BS = 2048


def _mlp_body(u_ref, i_ref, w1u_ref, w1i_ref, b1_ref, w2t_ref, b2_ref, o_ref):
    h = (
        jnp.dot(u_ref[...], w1u_ref[...], preferred_element_type=jnp.float32)
        + jnp.dot(i_ref[...], w1i_ref[...], preferred_element_type=jnp.float32)
        + b1_ref[...]
    )
    h = jnp.maximum(h, 0.0)
    o_ref[...] = jnp.sum(h * w2t_ref[...], axis=1, keepdims=True) + b2_ref[...]


def _mlp(u_emb, i_emb, W1u, W1i, b1, W2t, b2):
    return pl.pallas_call(
        _mlp_body,
        grid=(B // BS,),
        in_specs=[
            pl.BlockSpec((BS, NF), lambda i: (i, 0)),
            pl.BlockSpec((BS, NF), lambda i: (i, 0)),
            pl.BlockSpec((NF, H1), lambda i: (0, 0)),
            pl.BlockSpec((NF, H1), lambda i: (0, 0)),
            pl.BlockSpec((1, H1), lambda i: (0, 0)),
            pl.BlockSpec((1, H1), lambda i: (0, 0)),
            pl.BlockSpec((1, 1), lambda i: (0, 0)),
        ],
        out_specs=pl.BlockSpec((BS, 1), lambda i: (i, 0)),
        out_shape=jax.ShapeDtypeStruct((B, 1), jnp.float32),
    )(u_emb, i_emb, W1u, W1i, b1, W2t, b2)


@jax.jit
def kernel(users, items, user_table, item_table, W1, b1, W2, b2):
    users2 = users.reshape(NW, BPW)
    items2 = items.reshape(NW, BPW)
    u_emb, i_emb = _sc_gather(users2, items2, user_table, item_table)
    W1u = W1[:NF]
    W1i = W1[NF:]
    return _mlp(u_emb, i_emb, W1u, W1i,
                b1.reshape(1, H1), W2.reshape(1, H1), b2.reshape(1, 1))
